# baseline (speedup 1.0000x reference)
import functools

import jax
import jax.numpy as jnp
from jax import lax
from jax.experimental import pallas as pl
from jax.experimental.pallas import tpu as pltpu
from jax.experimental.pallas import tpu_sc as plsc

_VOXEL_SIZE = 0.05
_NUM_DISTS = 512
_GRID = 21
_V = _GRID ** 3
_VPAD = 9264
_NUM_CH = 10
_LANES = 16


def _bin_body(xt_ref, flat_ref):
    pts = xt_ref[0]
    mn = jnp.min(pts, axis=1, keepdims=True)
    vox = jnp.clip(jnp.floor((pts - mn) / _VOXEL_SIZE).astype(jnp.int32),
                   0, _GRID - 1)
    flat_ref[0] = (vox[0:1] * (_GRID * _GRID) + vox[1:2] * _GRID + vox[2:3])


_PAIR_ROWS = [(0, 0, 0), (1, 2, 0), (0, 1, 0), (0, 2, 1), (1, 2, 0)]
_PAIR_VALS = [
    (None, "a"), ("a", "b"), ("aa", "ab"), ("ab", "cc"), ("ab", "bb")]
_UNROLL = 8


def _sc_body(nbatch, xt_hbm, flat_hbm, sidx_hbm, zeros_hbm, out_hbm,
             flat_v, ra_v, rb_v, rc_v, t0_v, t1_v, bm_v, sidx_v, nb_v,
             p_v, f_v, shared_sp, sem):
    n = flat_v.shape[0]
    k = sidx_v.shape[0]
    wid = lax.axis_index("c") * 16 + lax.axis_index("s")

    for t in range(nbatch * 5):
        b, p = t // 5, t % 5
        w = 16 * (b // 2) + 5 * (b % 2) + p
        da, db, dc = _PAIR_ROWS[p]
        e0, e1 = _PAIR_VALS[p]
        used = set((e0 or "") + (e1 or ""))

        @pl.when(wid == w)
        def _task(b=b, p=p, da=da, db=db, dc=dc, e0=e0, e1=e1, used=used):
            cps = [pltpu.async_copy(flat_hbm.at[b, 0], flat_v, sem),
                   pltpu.async_copy(zeros_hbm, t0_v, sem),
                   pltpu.async_copy(zeros_hbm, t1_v, sem),
                   pltpu.async_copy(zeros_hbm, bm_v, sem),
                   pltpu.async_copy(sidx_hbm.at[b], sidx_v, sem)]
            if "a" in used:
                cps.append(pltpu.async_copy(xt_hbm.at[b, da], ra_v, sem))
            if "b" in used:
                cps.append(pltpu.async_copy(xt_hbm.at[b, db], rb_v, sem))
            if "c" in used:
                cps.append(pltpu.async_copy(xt_hbm.at[b, dc], rc_v, sem))
            for cp in cps:
                cp.wait()

            ones = jnp.ones((_LANES,), jnp.float32)

            def bb_(i, carry):
                s = pl.ds(i * _LANES, _LANES)
                nb = plsc.load_gather(flat_v, [sidx_v[s]])
                nb_v[s] = nb
                plsc.store_scatter(bm_v, [nb], ones)
                return carry
            lax.fori_loop(0, k // _LANES, bb_, 0)

            def val(expr, s):
                if expr is None:
                    return ones
                rows = {"a": ra_v, "b": rb_v, "c": rc_v}
                if len(expr) == 1:
                    return rows[expr][s]
                return rows[expr[0]][s] * rows[expr[1]][s]

            def sb(i, carry):
                for u in range(_UNROLL):
                    s = pl.ds((i * _UNROLL + u) * _LANES, _LANES)
                    idx = flat_v[s]
                    msk = plsc.load_gather(bm_v, [idx]) > 0.0
                    plsc.addupdate_scatter(t0_v, [idx], val(e0, s), mask=msk)
                    plsc.addupdate_scatter(t1_v, [idx], val(e1, s), mask=msk)
                return carry
            lax.fori_loop(0, n // (_LANES * _UNROLL), sb, 0)

            def gb(i, carry):
                s = pl.ds(i * _LANES, _LANES)
                nb = nb_v[s]
                p_v[2 * p, s] = plsc.load_gather(t0_v, [nb])
                p_v[2 * p + 1, s] = plsc.load_gather(t1_v, [nb])
                return carry
            lax.fori_loop(0, k // _LANES, gb, 0)

            pltpu.sync_copy(p_v.at[2 * p], shared_sp.at[b % 2, 2 * p])
            pltpu.sync_copy(p_v.at[2 * p + 1], shared_sp.at[b % 2, 2 * p + 1])

    plsc.subcore_barrier()

    for b in range(nbatch):
        w = 16 * (b // 2) + 10 + (b % 2)

        @pl.when(wid == w)
        def _fin(b=b):
            pltpu.sync_copy(shared_sp.at[b % 2], p_v)
            lane = lax.iota(jnp.int32, _LANES) * 12

            def fb(i, carry):
                s = pl.ds(i * _LANES, _LANES)
                inv = 1.0 / jnp.maximum(p_v[0, s], 1.0)
                m0 = p_v[1, s] * inv
                m1 = p_v[2, s] * inv
                m2 = p_v[3, s] * inv
                c00 = p_v[4, s] * inv - m0 * m0
                c01 = p_v[5, s] * inv - m0 * m1
                c02 = p_v[6, s] * inv - m0 * m2
                c11 = p_v[7, s] * inv - m1 * m1
                c12 = p_v[8, s] * inv - m1 * m2
                c22 = p_v[9, s] * inv - m2 * m2
                base = lane + i * (_LANES * 12)
                outs = [m0, m1, m2, c00, c01, c02, c01, c11, c12,
                        c02, c12, c22]
                for ch, v in enumerate(outs):
                    plsc.store_scatter(f_v, [base + ch], v)
                return carry
            lax.fori_loop(0, k // _LANES, fb, 0)

            pltpu.sync_copy(f_v, out_hbm.at[b])


def kernel(x):
    B, N, _ = x.shape
    K = _NUM_DISTS

    def _make_sidx():
        skeys = jax.random.split(jax.random.key(42), B)
        return jax.vmap(
            lambda kk: jax.random.permutation(kk, N)[:K])(skeys)

    try:
        with jax.ensure_compile_time_eval():
            sampled_idx = _make_sidx()
    except Exception:
        sampled_idx = _make_sidx()
    sidx = sampled_idx.astype(jnp.int32)
    zeros = jnp.zeros((_VPAD,), jnp.float32)

    xt = jnp.transpose(x, (0, 2, 1))

    flat = pl.pallas_call(
        _bin_body,
        grid=(B,),
        in_specs=[pl.BlockSpec((1, 3, N), lambda b: (b, 0, 0))],
        out_specs=pl.BlockSpec((1, 1, N), lambda b: (b, 0, 0)),
        out_shape=jax.ShapeDtypeStruct((B, 1, N), jnp.int32),
    )(xt)

    mesh = plsc.VectorSubcoreMesh(core_axis_name="c", subcore_axis_name="s",
                                  num_cores=2, num_subcores=16)
    outf = pl.kernel(
        functools.partial(_sc_body, B),
        out_type=jax.ShapeDtypeStruct((B, K * 12), jnp.float32),
        mesh=mesh,
        compiler_params=pltpu.CompilerParams(needs_layout_passes=False,
                                             use_tc_tiling_on_sc=False),
        scratch_types=[
            pltpu.VMEM((N,), jnp.int32),
            pltpu.VMEM((N,), jnp.float32),
            pltpu.VMEM((N,), jnp.float32),
            pltpu.VMEM((N,), jnp.float32),
            pltpu.VMEM((_VPAD,), jnp.float32),
            pltpu.VMEM((_VPAD,), jnp.float32),
            pltpu.VMEM((_VPAD,), jnp.float32),
            pltpu.VMEM((K,), jnp.int32),
            pltpu.VMEM((K,), jnp.int32),
            pltpu.VMEM((_NUM_CH, K), jnp.float32),
            pltpu.VMEM((K * 12,), jnp.float32),
            pltpu.VMEM_SHARED((2, _NUM_CH, K), jnp.float32),
            pltpu.SemaphoreType.DMA,
        ],
    )(xt, flat, sidx, zeros)

    return outf.reshape(B, K, 12)


# speedup vs baseline: 1.1574x; 1.1574x over previous
import functools

import jax
import jax.numpy as jnp
from jax import lax
from jax.experimental import pallas as pl
from jax.experimental.pallas import tpu as pltpu
from jax.experimental.pallas import tpu_sc as plsc

_VOXEL_SIZE = 0.05
_NUM_DISTS = 512
_GRID = 21
_V = _GRID ** 3
_VPAD = 9264
_NUM_CH = 10
_LANES = 16


def _bin_body(xt_ref, flat_ref):
    pts = xt_ref[0]
    mn = jnp.min(pts, axis=1, keepdims=True)
    vox = jnp.clip(jnp.floor((pts - mn) / _VOXEL_SIZE).astype(jnp.int32),
                   0, _GRID - 1)
    flat_ref[0] = (vox[0:1] * (_GRID * _GRID) + vox[1:2] * _GRID + vox[2:3])


_PAIR_ROWS = [(0, 0, 0), (1, 2, 0), (0, 1, 0), (0, 2, 1), (1, 2, 0)]
_PAIR_VALS = [
    (None, "a"), ("a", "b"), ("aa", "ab"), ("ab", "cc"), ("ab", "bb")]
_UNROLL = 8


def _sc_body(nbatch, xt_hbm, flat_hbm, sidx_hbm, zeros_hbm, out_hbm,
             flat_v, ra_v, rb_v, rc_v, t0_v, t1_v, sidx_v,
             p_v, f_v, shared_sp, sem):
    n = flat_v.shape[0]
    k = sidx_v.shape[0]
    wid = lax.axis_index("c") * 16 + lax.axis_index("s")

    for t in range(nbatch * 5):
        b, p = t // 5, t % 5
        w = 16 * (b // 2) + 5 * (b % 2) + p
        da, db, dc = _PAIR_ROWS[p]
        e0, e1 = _PAIR_VALS[p]
        used = set((e0 or "") + (e1 or ""))

        @pl.when(wid == w)
        def _task(b=b, p=p, da=da, db=db, dc=dc, e0=e0, e1=e1, used=used):
            cps = [pltpu.async_copy(flat_hbm.at[b, 0], flat_v, sem),
                   pltpu.async_copy(zeros_hbm, t0_v, sem),
                   pltpu.async_copy(zeros_hbm, t1_v, sem),
                   pltpu.async_copy(sidx_hbm.at[b], sidx_v, sem)]
            if "a" in used:
                cps.append(pltpu.async_copy(xt_hbm.at[b, da], ra_v, sem))
            if "b" in used:
                cps.append(pltpu.async_copy(xt_hbm.at[b, db], rb_v, sem))
            if "c" in used:
                cps.append(pltpu.async_copy(xt_hbm.at[b, dc], rc_v, sem))
            for cp in cps:
                cp.wait()

            ones = jnp.ones((_LANES,), jnp.float32)

            def val(expr, s):
                if expr is None:
                    return ones
                rows = {"a": ra_v, "b": rb_v, "c": rc_v}
                if len(expr) == 1:
                    return rows[expr][s]
                return rows[expr[0]][s] * rows[expr[1]][s]

            def sb(i, carry):
                for u in range(_UNROLL):
                    s = pl.ds((i * _UNROLL + u) * _LANES, _LANES)
                    idx = flat_v[s]
                    plsc.addupdate_scatter(t0_v, [idx], val(e0, s))
                    plsc.addupdate_scatter(t1_v, [idx], val(e1, s))
                return carry
            lax.fori_loop(0, n // (_LANES * _UNROLL), sb, 0)

            def gb(i, carry):
                s = pl.ds(i * _LANES, _LANES)
                nb = plsc.load_gather(flat_v, [sidx_v[s]])
                p_v[2 * p, s] = plsc.load_gather(t0_v, [nb])
                p_v[2 * p + 1, s] = plsc.load_gather(t1_v, [nb])
                return carry
            lax.fori_loop(0, k // _LANES, gb, 0)

            pltpu.sync_copy(p_v.at[2 * p], shared_sp.at[b % 2, 2 * p])
            pltpu.sync_copy(p_v.at[2 * p + 1], shared_sp.at[b % 2, 2 * p + 1])

    plsc.subcore_barrier()

    for b in range(nbatch):
        w = 16 * (b // 2) + 10 + (b % 2)

        @pl.when(wid == w)
        def _fin(b=b):
            pltpu.sync_copy(shared_sp.at[b % 2], p_v)
            lane = lax.iota(jnp.int32, _LANES) * 12

            def fb(i, carry):
                s = pl.ds(i * _LANES, _LANES)
                inv = 1.0 / jnp.maximum(p_v[0, s], 1.0)
                m0 = p_v[1, s] * inv
                m1 = p_v[2, s] * inv
                m2 = p_v[3, s] * inv
                c00 = p_v[4, s] * inv - m0 * m0
                c01 = p_v[5, s] * inv - m0 * m1
                c02 = p_v[6, s] * inv - m0 * m2
                c11 = p_v[7, s] * inv - m1 * m1
                c12 = p_v[8, s] * inv - m1 * m2
                c22 = p_v[9, s] * inv - m2 * m2
                base = lane + i * (_LANES * 12)
                outs = [m0, m1, m2, c00, c01, c02, c01, c11, c12,
                        c02, c12, c22]
                for ch, v in enumerate(outs):
                    plsc.store_scatter(f_v, [base + ch], v)
                return carry
            lax.fori_loop(0, k // _LANES, fb, 0)

            pltpu.sync_copy(f_v, out_hbm.at[b])


def kernel(x):
    B, N, _ = x.shape
    K = _NUM_DISTS

    def _make_sidx():
        skeys = jax.random.split(jax.random.key(42), B)
        return jax.vmap(
            lambda kk: jax.random.permutation(kk, N)[:K])(skeys)

    try:
        with jax.ensure_compile_time_eval():
            sampled_idx = _make_sidx()
    except Exception:
        sampled_idx = _make_sidx()
    sidx = sampled_idx.astype(jnp.int32)
    zeros = jnp.zeros((_VPAD,), jnp.float32)

    xt = jnp.transpose(x, (0, 2, 1))

    flat = pl.pallas_call(
        _bin_body,
        grid=(B,),
        in_specs=[pl.BlockSpec((1, 3, N), lambda b: (b, 0, 0))],
        out_specs=pl.BlockSpec((1, 1, N), lambda b: (b, 0, 0)),
        out_shape=jax.ShapeDtypeStruct((B, 1, N), jnp.int32),
    )(xt)

    mesh = plsc.VectorSubcoreMesh(core_axis_name="c", subcore_axis_name="s",
                                  num_cores=2, num_subcores=16)
    outf = pl.kernel(
        functools.partial(_sc_body, B),
        out_type=jax.ShapeDtypeStruct((B, K * 12), jnp.float32),
        mesh=mesh,
        compiler_params=pltpu.CompilerParams(needs_layout_passes=False,
                                             use_tc_tiling_on_sc=False),
        scratch_types=[
            pltpu.VMEM((N,), jnp.int32),
            pltpu.VMEM((N,), jnp.float32),
            pltpu.VMEM((N,), jnp.float32),
            pltpu.VMEM((N,), jnp.float32),
            pltpu.VMEM((_VPAD,), jnp.float32),
            pltpu.VMEM((_VPAD,), jnp.float32),
            pltpu.VMEM((K,), jnp.int32),
            pltpu.VMEM((_NUM_CH, K), jnp.float32),
            pltpu.VMEM((K * 12,), jnp.float32),
            pltpu.VMEM_SHARED((2, _NUM_CH, K), jnp.float32),
            pltpu.SemaphoreType.DMA,
        ],
    )(xt, flat, sidx, zeros)

    return outf.reshape(B, K, 12)


# speedup vs baseline: 1.3212x; 1.1414x over previous
import functools

import jax
import jax.numpy as jnp
from jax import lax
from jax.experimental import pallas as pl
from jax.experimental.pallas import tpu as pltpu
from jax.experimental.pallas import tpu_sc as plsc

_VOXEL_SIZE = 0.05
_NUM_DISTS = 512
_GRID = 21
_V = _GRID ** 3
_VPAD = 9264
_NUM_CH = 10
_LANES = 16


def _bin_body(xt_ref, flat_ref):
    pts = xt_ref[0]
    mn = jnp.min(pts, axis=1, keepdims=True)
    vox = jnp.clip(jnp.floor((pts - mn) / _VOXEL_SIZE).astype(jnp.int32),
                   0, _GRID - 1)
    flat_ref[0] = (vox[0:1] * (_GRID * _GRID) + vox[1:2] * _GRID + vox[2:3])


_PAIR_ROWS = [(0, 0, 0), (1, 2, 0), (0, 1, 0), (0, 2, 1), (1, 2, 0)]
_PAIR_VALS = [
    (None, "a"), ("a", "b"), ("aa", "ab"), ("ab", "cc"), ("ab", "bb")]
_UNROLL = 8


def _sc_body(nbatch, xt_hbm, flat_hbm, sidx_hbm, zeros_hbm, out_hbm,
             flat_v, ra_v, rb_v, rc_v, t0_v, t1_v, sidx_v,
             p_v, f_v, shared_sp, sem):
    n = flat_v.shape[0]
    k = sidx_v.shape[0]
    wid = lax.axis_index("c") * 16 + lax.axis_index("s")

    for t in range(nbatch * 5):
        b, p = t // 5, t % 5
        w = 16 * (b // 2) + 5 * (b % 2) + p
        da, db, dc = _PAIR_ROWS[p]
        e0, e1 = _PAIR_VALS[p]
        used = set((e0 or "") + (e1 or ""))

        @pl.when(wid == w)
        def _task(b=b, p=p, da=da, db=db, dc=dc, e0=e0, e1=e1, used=used):
            cps = [pltpu.async_copy(flat_hbm.at[b, 0], flat_v, sem),
                   pltpu.async_copy(zeros_hbm, t0_v, sem),
                   pltpu.async_copy(zeros_hbm, t1_v, sem),
                   pltpu.async_copy(sidx_hbm.at[b], sidx_v, sem)]
            if "a" in used:
                cps.append(pltpu.async_copy(xt_hbm.at[b, da], ra_v, sem))
            if "b" in used:
                cps.append(pltpu.async_copy(xt_hbm.at[b, db], rb_v, sem))
            if "c" in used:
                cps.append(pltpu.async_copy(xt_hbm.at[b, dc], rc_v, sem))
            for cp in cps:
                cp.wait()

            ones = jnp.ones((_LANES,), jnp.float32)

            def val(expr, s):
                if expr is None:
                    return ones
                rows = {"a": ra_v, "b": rb_v, "c": rc_v}
                if len(expr) == 1:
                    return rows[expr][s]
                return rows[expr[0]][s] * rows[expr[1]][s]

            @plsc.parallel_loop(0, n // _LANES, unroll=_UNROLL)
            def sb(i):
                s = pl.ds(i * _LANES, _LANES)
                idx = flat_v[s]
                plsc.addupdate_scatter(t0_v, [idx], val(e0, s))
                plsc.addupdate_scatter(t1_v, [idx], val(e1, s))

            def gb(i, carry):
                s = pl.ds(i * _LANES, _LANES)
                nb = plsc.load_gather(flat_v, [sidx_v[s]])
                p_v[2 * p, s] = plsc.load_gather(t0_v, [nb])
                p_v[2 * p + 1, s] = plsc.load_gather(t1_v, [nb])
                return carry
            lax.fori_loop(0, k // _LANES, gb, 0)

            pltpu.sync_copy(p_v.at[2 * p], shared_sp.at[b % 2, 2 * p])
            pltpu.sync_copy(p_v.at[2 * p + 1], shared_sp.at[b % 2, 2 * p + 1])

    plsc.subcore_barrier()

    for b in range(nbatch):
        w = 16 * (b // 2) + 10 + (b % 2)

        @pl.when(wid == w)
        def _fin(b=b):
            pltpu.sync_copy(shared_sp.at[b % 2], p_v)
            lane = lax.iota(jnp.int32, _LANES) * 12

            def fb(i, carry):
                s = pl.ds(i * _LANES, _LANES)
                inv = 1.0 / jnp.maximum(p_v[0, s], 1.0)
                m0 = p_v[1, s] * inv
                m1 = p_v[2, s] * inv
                m2 = p_v[3, s] * inv
                c00 = p_v[4, s] * inv - m0 * m0
                c01 = p_v[5, s] * inv - m0 * m1
                c02 = p_v[6, s] * inv - m0 * m2
                c11 = p_v[7, s] * inv - m1 * m1
                c12 = p_v[8, s] * inv - m1 * m2
                c22 = p_v[9, s] * inv - m2 * m2
                base = lane + i * (_LANES * 12)
                outs = [m0, m1, m2, c00, c01, c02, c01, c11, c12,
                        c02, c12, c22]
                for ch, v in enumerate(outs):
                    plsc.store_scatter(f_v, [base + ch], v)
                return carry
            lax.fori_loop(0, k // _LANES, fb, 0)

            pltpu.sync_copy(f_v, out_hbm.at[b])


def kernel(x):
    B, N, _ = x.shape
    K = _NUM_DISTS

    def _make_sidx():
        skeys = jax.random.split(jax.random.key(42), B)
        return jax.vmap(
            lambda kk: jax.random.permutation(kk, N)[:K])(skeys)

    try:
        with jax.ensure_compile_time_eval():
            sampled_idx = _make_sidx()
    except Exception:
        sampled_idx = _make_sidx()
    sidx = sampled_idx.astype(jnp.int32)
    zeros = jnp.zeros((_VPAD,), jnp.float32)

    xt = jnp.transpose(x, (0, 2, 1))

    flat = pl.pallas_call(
        _bin_body,
        grid=(B,),
        in_specs=[pl.BlockSpec((1, 3, N), lambda b: (b, 0, 0))],
        out_specs=pl.BlockSpec((1, 1, N), lambda b: (b, 0, 0)),
        out_shape=jax.ShapeDtypeStruct((B, 1, N), jnp.int32),
    )(xt)

    mesh = plsc.VectorSubcoreMesh(core_axis_name="c", subcore_axis_name="s",
                                  num_cores=2, num_subcores=16)
    outf = pl.kernel(
        functools.partial(_sc_body, B),
        out_type=jax.ShapeDtypeStruct((B, K * 12), jnp.float32),
        mesh=mesh,
        compiler_params=pltpu.CompilerParams(needs_layout_passes=False,
                                             use_tc_tiling_on_sc=False),
        scratch_types=[
            pltpu.VMEM((N,), jnp.int32),
            pltpu.VMEM((N,), jnp.float32),
            pltpu.VMEM((N,), jnp.float32),
            pltpu.VMEM((N,), jnp.float32),
            pltpu.VMEM((_VPAD,), jnp.float32),
            pltpu.VMEM((_VPAD,), jnp.float32),
            pltpu.VMEM((K,), jnp.int32),
            pltpu.VMEM((_NUM_CH, K), jnp.float32),
            pltpu.VMEM((K * 12,), jnp.float32),
            pltpu.VMEM_SHARED((2, _NUM_CH, K), jnp.float32),
            pltpu.SemaphoreType.DMA,
        ],
    )(xt, flat, sidx, zeros)

    return outf.reshape(B, K, 12)
